# baseline (speedup 1.0000x reference)
import functools

import jax
import jax.numpy as jnp
from jax import lax
from jax.experimental import pallas as pl
from jax.experimental.pallas import tpu as pltpu
from jax.experimental.pallas import tpu_sc as plsc

T = 2048
D = 1024
F = 512
E = 8
RSF = 2.5
EPS = 1e-6

TBLK = 128
BLK = 128
NP = 2 * T
P = NP + E * BLK
NT = P // BLK
CH = 256
NW = 32
TPW = T // NW


def _pre_body(x_ref, rw_ref, swg_ref, swu_ref, swd_ref, ln_ref,
              h_ref, sh_ref, lg_ref):
    x = x_ref[...]
    var = jnp.mean(x * x, axis=-1, keepdims=True)
    h = x * lax.rsqrt(var + EPS) * ln_ref[...]
    h_ref[...] = h
    lg_ref[...] = jnp.dot(h, rw_ref[...], preferred_element_type=jnp.float32)
    sg = jnp.dot(h, swg_ref[...], preferred_element_type=jnp.float32)
    su = jnp.dot(h, swu_ref[...], preferred_element_type=jnp.float32)
    sh_ref[...] = jnp.dot(jax.nn.silu(sg) * su, swd_ref[...],
                          preferred_element_type=jnp.float32)


def _route_body(lg_ref, bias_ref, pos1_ref, pos2_ref, w_ref, te_ref):
    logits = lg_ref[...]
    scores = jax.nn.sigmoid(logits)
    sfc = scores + bias_ref[...]

    eidx = lax.broadcasted_iota(jnp.int32, (T, E), 1)
    neg = jnp.float32(-jnp.inf)
    m1 = jnp.max(sfc, axis=1, keepdims=True)
    i1 = jnp.min(jnp.where(sfc == m1, eidx, E), axis=1, keepdims=True)
    sfc2 = jnp.where(eidx == i1, neg, sfc)
    m2 = jnp.max(sfc2, axis=1, keepdims=True)
    i2 = jnp.min(jnp.where(sfc2 == m2, eidx, E), axis=1, keepdims=True)

    w1 = jnp.sum(jnp.where(eidx == i1, scores, 0.0), axis=1, keepdims=True)
    w2 = jnp.sum(jnp.where(eidx == i2, scores, 0.0), axis=1, keepdims=True)
    denom = w1 + w2 + 1e-20
    wa = w1 / denom * RSF
    wb = w2 / denom * RSF
    w_ref[...] = jnp.concatenate(
        [jnp.broadcast_to(wa, (T, 8)), jnp.broadcast_to(wb, (T, 8))], axis=1)

    onehot = (jnp.where(eidx == i1, 1.0, 0.0)
              + jnp.where(eidx == i2, 1.0, 0.0))

    rank1, rank2 = [], []
    colg = lax.broadcasted_iota(jnp.int32, (CH, T), 1)
    rowg = lax.broadcasted_iota(jnp.int32, (CH, T), 0)
    for c in range(T // CH):
        mask = jnp.where(colg < rowg + (c * CH), 1.0, 0.0)
        cx = jnp.dot(mask, onehot, preferred_element_type=jnp.float32)
        sl = slice(c * CH, (c + 1) * CH)
        rank1.append(jnp.sum(jnp.where(eidx[sl] == i1[sl], cx, 0.0),
                             axis=1, keepdims=True))
        rank2.append(jnp.sum(jnp.where(eidx[sl] == i2[sl], cx, 0.0),
                             axis=1, keepdims=True))
    rank1 = jnp.concatenate(rank1, axis=0)
    rank2 = jnp.concatenate(rank2, axis=0)

    ones_row = jnp.ones((1, T), jnp.float32)
    counts_row = jnp.dot(ones_row, onehot, preferred_element_type=jnp.float32)
    padded_row = jnp.floor((counts_row + (BLK - 1)) / BLK) * BLK
    er = lax.broadcasted_iota(jnp.int32, (E, E), 0)
    ec = lax.broadcasted_iota(jnp.int32, (E, E), 1)
    strict = jnp.where(er < ec, 1.0, 0.0)
    start_row = jnp.dot(padded_row, strict, preferred_element_type=jnp.float32)

    s1 = jnp.sum(jnp.where(eidx == i1, start_row, 0.0), axis=1, keepdims=True)
    s2 = jnp.sum(jnp.where(eidx == i2, start_row, 0.0), axis=1, keepdims=True)
    pos1_ref[...] = (s1 + rank1).astype(jnp.int32)
    pos2_ref[...] = (s2 + rank2).astype(jnp.int32)

    ones_col = jnp.ones((T, 1), jnp.float32)
    counts_col = lax.dot_general(onehot, ones_col, (((0,), (0,)), ((), ())))
    padded_col = jnp.floor((counts_col + (BLK - 1)) / BLK) * BLK
    strict_lo = jnp.where(ec < er, 1.0, 0.0)
    start_col = jnp.dot(strict_lo, padded_col, preferred_element_type=jnp.float32)
    jrow = lax.broadcasted_iota(jnp.int32, (E, 64), 1) * BLK
    eid = lax.broadcasted_iota(jnp.int32, (E, 64), 0).astype(jnp.float32)
    start_i = start_col.astype(jnp.int32)
    padded_i = padded_col.astype(jnp.int32)
    hit = jnp.where((start_i <= jrow) & (jrow < start_i + padded_i),
                    eid, 0.0)
    te = jnp.sum(hit, axis=0, keepdims=True)
    te_ref[...] = jnp.broadcast_to(te, (8, 64)).astype(jnp.int32)


def _gemm_body(te_ref, x_ref, wg_ref, wu_ref, wd_ref, y_ref):
    del te_ref
    x = x_ref[...]
    a1 = jnp.dot(x, wg_ref[0], preferred_element_type=jnp.float32)
    a2 = jnp.dot(x, wu_ref[0], preferred_element_type=jnp.float32)
    inter = jax.nn.silu(a1) * a2
    y_ref[...] = jnp.dot(inter, wd_ref[0], preferred_element_type=jnp.float32)


def _combine_body(sh_ref, y1_ref, y2_ref, w_ref, out_ref):
    w = w_ref[...]
    out_ref[...] = (sh_ref[...]
                    + y1_ref[...] * w[:, 0:1]
                    + y2_ref[...] * w[:, 8:9])


@functools.cache
def _sc_mesh():
    return plsc.VectorSubcoreMesh(core_axis_name="c", subcore_axis_name="s")


def _scatter_rows(h_hbm, p1_hbm, p2_hbm, xs_hbm, rows_v, idx_v, sem):
    wid = lax.axis_index("s") * 2 + lax.axis_index("c")
    base = wid * TPW
    pltpu.sync_copy(h_hbm.at[pl.ds(base, TPW)], rows_v)
    pltpu.sync_copy(p1_hbm.at[pl.ds(base, TPW)], idx_v)
    pltpu.async_copy(rows_v, xs_hbm.at[idx_v], sem).wait()
    pltpu.sync_copy(p2_hbm.at[pl.ds(base, TPW)], idx_v)
    pltpu.async_copy(rows_v, xs_hbm.at[idx_v], sem).wait()


def _gather_rows(ys_hbm, p1_hbm, p2_hbm, y1_hbm, y2_hbm, rows_v, idx_v, sem):
    wid = lax.axis_index("s") * 2 + lax.axis_index("c")
    base = wid * TPW
    pltpu.sync_copy(p1_hbm.at[pl.ds(base, TPW)], idx_v)
    pltpu.async_copy(ys_hbm.at[idx_v], rows_v, sem).wait()
    pltpu.sync_copy(rows_v, y1_hbm.at[pl.ds(base, TPW)])
    pltpu.sync_copy(p2_hbm.at[pl.ds(base, TPW)], idx_v)
    pltpu.async_copy(ys_hbm.at[idx_v], rows_v, sem).wait()
    pltpu.sync_copy(rows_v, y2_hbm.at[pl.ds(base, TPW)])


@jax.jit
def kernel(hidden_states, router_w, expert_bias, w_gate, w_up, w_down,
           sw_gate, sw_up, sw_down, ln_w):
    f32 = jnp.float32
    full = lambda *s: pl.BlockSpec(s, lambda i: (0,) * len(s))

    h, shared, logits = pl.pallas_call(
        _pre_body,
        grid=(T // TBLK,),
        in_specs=[
            pl.BlockSpec((TBLK, D), lambda i: (i, 0)),
            full(D, E), full(D, F), full(D, F), full(F, D), full(1, D),
        ],
        out_specs=[
            pl.BlockSpec((TBLK, D), lambda i: (i, 0)),
            pl.BlockSpec((TBLK, D), lambda i: (i, 0)),
            pl.BlockSpec((TBLK, E), lambda i: (i, 0)),
        ],
        out_shape=[
            jax.ShapeDtypeStruct((T, D), f32),
            jax.ShapeDtypeStruct((T, D), f32),
            jax.ShapeDtypeStruct((T, E), f32),
        ],
    )(hidden_states, router_w, sw_gate, sw_up, sw_down, ln_w.reshape(1, D))

    pos1, pos2, w, te = pl.pallas_call(
        _route_body,
        grid=(1,),
        in_specs=[full(T, E), full(1, E)],
        out_specs=[full(T, 1), full(T, 1), full(T, 16), full(8, 64)],
        out_shape=[
            jax.ShapeDtypeStruct((T, 1), jnp.int32),
            jax.ShapeDtypeStruct((T, 1), jnp.int32),
            jax.ShapeDtypeStruct((T, 16), f32),
            jax.ShapeDtypeStruct((8, 64), jnp.int32),
        ],
    )(logits, expert_bias.reshape(1, E))

    pos1f = pos1.reshape(T)
    pos2f = pos2.reshape(T)
    te_flat = te[0, :NT]

    scatter = functools.partial(
        pl.kernel, mesh=_sc_mesh(),
        out_type=jax.ShapeDtypeStruct((P, D), f32),
        scratch_types=[
            pltpu.VMEM((TPW, D), f32),
            pltpu.VMEM((TPW,), jnp.int32),
            pltpu.SemaphoreType.DMA,
        ],
    )(_scatter_rows)
    x_sorted = scatter(h, pos1f, pos2f)

    y_sorted = pl.pallas_call(
        _gemm_body,
        grid_spec=pltpu.PrefetchScalarGridSpec(
            num_scalar_prefetch=1,
            grid=(NT,),
            in_specs=[
                pl.BlockSpec((BLK, D), lambda i, te: (i, 0)),
                pl.BlockSpec((1, D, F), lambda i, te: (te[i], 0, 0)),
                pl.BlockSpec((1, D, F), lambda i, te: (te[i], 0, 0)),
                pl.BlockSpec((1, F, D), lambda i, te: (te[i], 0, 0)),
            ],
            out_specs=pl.BlockSpec((BLK, D), lambda i, te: (i, 0)),
        ),
        out_shape=jax.ShapeDtypeStruct((P, D), f32),
    )(te_flat, x_sorted, w_gate, w_up, w_down)

    gather = functools.partial(
        pl.kernel, mesh=_sc_mesh(),
        out_type=[
            jax.ShapeDtypeStruct((T, D), f32),
            jax.ShapeDtypeStruct((T, D), f32),
        ],
        scratch_types=[
            pltpu.VMEM((TPW, D), f32),
            pltpu.VMEM((TPW,), jnp.int32),
            pltpu.SemaphoreType.DMA,
        ],
    )(_gather_rows)
    y1, y2 = gather(y_sorted, pos1f, pos2f)

    out = pl.pallas_call(
        _combine_body,
        grid=(T // TBLK,),
        in_specs=[
            pl.BlockSpec((TBLK, D), lambda i: (i, 0)),
            pl.BlockSpec((TBLK, D), lambda i: (i, 0)),
            pl.BlockSpec((TBLK, D), lambda i: (i, 0)),
            pl.BlockSpec((TBLK, 16), lambda i: (i, 0)),
        ],
        out_specs=pl.BlockSpec((TBLK, D), lambda i: (i, 0)),
        out_shape=jax.ShapeDtypeStruct((T, D), f32),
    )(shared, y1, y2, w)
    return out
